# Initial kernel scaffold; baseline (speedup 1.0000x reference)
import jax
import jax.numpy as jnp
from jax.experimental import pallas as pl


def kernel(x, edge_index, edge_attr, Wn0, bn0, We0, be0, Wp1_0, bp1_0, Wp2_0, bp2_0, Wn1, bn1, We1, be1, Wp1_1, bp1_1, Wp2_1, bp2_1):
    raise NotImplementedError("write your pallas kernel here")


# speedup vs baseline: 3.8250x; 3.8250x over previous
import functools

import jax
import jax.numpy as jnp
from jax import lax
from jax.experimental import pallas as pl
from jax.experimental.pallas import tpu as pltpu
from jax.experimental.pallas import tpu_sc as plsc

N = 50000
E = 1600000
HID = 32

NC = 2
NS = 16
NW = NC * NS

NP = 51200
NPT = NP // NS
B = 256
CHUNK = 128
NCH = B // CHUNK
EPW = 51200
EPAD = EPW * NW
NBLK = EPW // B


def _sc_layer_kernel(include_deg):
  mesh = plsc.VectorSubcoreMesh(core_axis_name="c", subcore_axis_name="s",
                                num_cores=NC, num_subcores=NS)

  out_type = [jax.ShapeDtypeStruct((NC, NP, HID), jnp.float32)]
  if include_deg:
    out_type.append(jax.ShapeDtypeStruct((NC, NP), jnp.float32))

  scratch = [
      pltpu.VMEM((NCH, CHUNK), jnp.int32),
      pltpu.VMEM((NCH, CHUNK), jnp.int32),
      pltpu.VMEM((B, HID), jnp.float32),
      pltpu.VMEM((B, HID), jnp.float32),
      pltpu.VMEM_SHARED((NP, HID), jnp.float32),
      pltpu.SemaphoreType.DMA,
      pltpu.SemaphoreType.DMA,
  ]
  if include_deg:
    scratch += [
        pltpu.VMEM((CHUNK,), jnp.float32),
        pltpu.VMEM_SHARED((NP,), jnp.float32),
        pltpu.SemaphoreType.DMA,
    ]

  def body(src2d, dst2d, eth, table, z32, *rest):
    if include_deg:
      (z1, aggr_out, deg_out, srcv, dstv, rows, erows,
       aggr_sh, gsem, ssem, onesv, deg_sh, dsem) = rest
    else:
      (aggr_out, srcv, dstv, rows, erows, aggr_sh, gsem, ssem) = rest

    c = lax.axis_index("c")
    s = lax.axis_index("s")
    wid = s * NC + c

    if include_deg:
      for i in range(CHUNK // 16):
        onesv[pl.ds(i * 16, 16)] = jnp.ones((16,), jnp.float32)

    pltpu.sync_copy(z32.at[pl.ds(s * NPT, NPT)],
                    aggr_sh.at[pl.ds(s * NPT, NPT)])
    if include_deg:
      pltpu.sync_copy(z1.at[pl.ds(s * NPT, NPT)],
                      deg_sh.at[pl.ds(s * NPT, NPT)])
    plsc.subcore_barrier()

    ebase = wid * EPW

    def block_body(blk, carry):
      base = pl.multiple_of(ebase + blk * B, B)
      row0 = pl.multiple_of(base // CHUNK, NCH)
      pltpu.sync_copy(src2d.at[pl.ds(row0, NCH)], srcv)
      pltpu.sync_copy(dst2d.at[pl.ds(row0, NCH)], dstv)

      ecp = pltpu.async_copy(eth.at[pl.ds(base, B)], erows, gsem)
      gcps = [
          pltpu.async_copy(table.at[srcv.at[j]],
                           rows.at[pl.ds(j * CHUNK, CHUNK)], gsem)
          for j in range(NCH)
      ]
      ecp.wait()
      for cp in gcps:
        cp.wait()

      scps = [
          pltpu.async_copy(rows.at[pl.ds(j * CHUNK, CHUNK)],
                           aggr_sh.at[dstv.at[j]], ssem, add=True)
          for j in range(NCH)
      ] + [
          pltpu.async_copy(erows.at[pl.ds(j * CHUNK, CHUNK)],
                           aggr_sh.at[dstv.at[j]], ssem, add=True)
          for j in range(NCH)
      ]
      for cp in scps:
        cp.wait()

      if include_deg:
        dcps = [
            pltpu.async_copy(onesv, deg_sh.at[dstv.at[j]], dsem, add=True)
            for j in range(NCH)
        ]
        for cp in dcps:
          cp.wait()
      return 0

    lax.fori_loop(0, NBLK, block_body, 0)

    plsc.subcore_barrier()

    pltpu.sync_copy(aggr_sh.at[pl.ds(s * NPT, NPT)],
                    aggr_out.at[c, pl.ds(s * NPT, NPT)])
    if include_deg:
      pltpu.sync_copy(deg_sh.at[pl.ds(s * NPT, NPT)],
                      deg_out.at[c, pl.ds(s * NPT, NPT)])

  return pl.kernel(body, out_type=out_type, mesh=mesh, scratch_types=scratch,
                   compiler_params=pltpu.CompilerParams(
                       needs_layout_passes=False,
                       use_tc_tiling_on_sc=False),
                   name="rgcn_sc_deg" if include_deg else "rgcn_sc")


@functools.lru_cache(maxsize=None)
def _get_sc_kernel(include_deg):
  return _sc_layer_kernel(include_deg)


RB = 1600
RBE = 6400


def _rb(minor, rb=RB):
  return pl.BlockSpec((rb, minor), lambda i: (i, 0))


def _full(shape):
  return pl.BlockSpec(shape, lambda i: tuple(0 for _ in shape))


def _tc_table_body(x_ref, wn_ref, bn_ref, out_ref):
  x = x_ref[...]
  wn = wn_ref[...]
  t = x[:, 0:1] * wn[0:1, :] + x[:, 1:2] * wn[1:2, :] + bn_ref[...]
  out_ref[...] = jnp.maximum(t, 0.0)


_tc_table = pl.pallas_call(
    _tc_table_body,
    grid=(NP // RB,),
    in_specs=[_rb(2), _full((2, HID)), _full((1, HID))],
    out_specs=_rb(HID),
    out_shape=jax.ShapeDtypeStruct((NP, HID), jnp.float32),
)


RBE2 = 3200


def _tc_et_body(a0_ref, a1_ref, w0r0_ref, w0r1_ref, b0_ref,
                w1r0_ref, w1r1_ref, b1_ref, et0_ref, et1_ref):
  a = a0_ref[...]
  b = a1_ref[...]
  et0_ref[...] = jnp.maximum(
      a * w0r0_ref[...] + b * w0r1_ref[...] + b0_ref[...], 0.0)
  et1_ref[...] = jnp.maximum(
      a * w1r0_ref[...] + b * w1r1_ref[...] + b1_ref[...], 0.0)


_tc_et = pl.pallas_call(
    _tc_et_body,
    grid=(EPAD // RBE2,),
    in_specs=[_rb(1, RBE2), _rb(1, RBE2),
              _full((1, HID)), _full((1, HID)), _full((1, HID)),
              _full((1, HID)), _full((1, HID)), _full((1, HID))],
    out_specs=[_rb(HID, RBE2), _rb(HID, RBE2)],
    out_shape=[
        jax.ShapeDtypeStruct((EPAD, HID), jnp.float32),
        jax.ShapeDtypeStruct((EPAD, HID), jnp.float32),
    ],
)


def _tc_post_body(x_ref, rn_ref, p0_ref, p1_ref, deg0_ref, deg1_ref,
                  wp1a_ref, wp1b_ref, bp1_ref, wp2_ref, bp2_ref,
                  wn_ref, bn_ref,
                  out_ref, rn_next_ref):
  x = x_ref[...]
  deg = deg0_ref[...] + deg1_ref[...]
  recip = 1.0 / jnp.maximum(deg, 1.0)
  aggr = (p0_ref[...] + p1_ref[...]) * recip
  t = aggr + rn_ref[...]
  wp1a = wp1a_ref[...]
  h = (x[:, 0:1] * wp1a[0:1, :] + x[:, 1:2] * wp1a[1:2, :]
       + jnp.dot(t, wp1b_ref[...], preferred_element_type=jnp.float32)
       + bp1_ref[...])
  h = jnp.maximum(h, 0.0)
  z = jnp.sum(h * wp2_ref[...], axis=1, keepdims=True)
  power = jax.nn.sigmoid(z + bp2_ref[...])
  out_ref[:, 0:1] = x[:, 0:1]
  out_ref[:, 1:2] = power
  wn = wn_ref[...]
  rn = x[:, 0:1] * wn[0:1, :] + power * wn[1:2, :] + bn_ref[...]
  rn_next_ref[...] = jnp.maximum(rn, 0.0)


_tc_post = pl.pallas_call(
    _tc_post_body,
    grid=(NP // RB,),
    in_specs=[_rb(2), _rb(HID), _rb(HID), _rb(HID), _rb(1), _rb(1),
              _full((2, 16)), _full((HID, 16)), _full((1, 16)),
              _full((1, 16)), _full((1, 1)),
              _full((2, HID)), _full((1, HID))],
    out_specs=[_rb(2), _rb(HID)],
    out_shape=[
        jax.ShapeDtypeStruct((NP, 2), jnp.float32),
        jax.ShapeDtypeStruct((NP, HID), jnp.float32),
    ],
)


def kernel(x, edge_index, edge_attr,
           Wn0, bn0, We0, be0, Wp1_0, bp1_0, Wp2_0, bp2_0,
           Wn1, bn1, We1, be1, Wp1_1, bp1_1, Wp2_1, bp2_1):
  src = edge_index[0]
  dst = edge_index[1]
  npad = EPAD - E
  pad_idx = (N + (jnp.arange(npad, dtype=jnp.int32) % (NP - N))).astype(jnp.int32)
  src_f = jnp.concatenate([src, pad_idx]).reshape(EPAD // CHUNK, CHUNK)
  dst_f = jnp.concatenate([dst, pad_idx]).reshape(EPAD // CHUNK, CHUNK)
  zpad = jnp.zeros((npad,), jnp.float32)
  a0p = jnp.concatenate([edge_attr[:, 0], zpad]).reshape(EPAD, 1)
  a1p = jnp.concatenate([edge_attr[:, 1], zpad]).reshape(EPAD, 1)
  xp = jnp.zeros((NP, 2), jnp.float32).at[:N].set(x)
  z32 = jnp.zeros((NP, HID), jnp.float32)
  z1 = jnp.zeros((NP,), jnp.float32)
  bn0r = bn0.reshape(1, HID)
  bn1r = bn1.reshape(1, HID)
  bp1_0r = bp1_0.reshape(1, 16)
  bp1_1r = bp1_1.reshape(1, 16)
  bp2_0r = bp2_0.reshape(1, 1)
  bp2_1r = bp2_1.reshape(1, 1)

  et0, et1 = _tc_et(a0p, a1p,
                    We0[0].reshape(1, HID), We0[1].reshape(1, HID),
                    be0.reshape(1, HID),
                    We1[0].reshape(1, HID), We1[1].reshape(1, HID),
                    be1.reshape(1, HID))
  rn0 = _tc_table(xp, Wn0, bn0r)

  aggr_p, deg_p = _get_sc_kernel(True)(src_f, dst_f, et0, rn0, z32, z1)
  deg0 = deg_p[0].reshape(NP, 1)
  deg1 = deg_p[1].reshape(NP, 1)
  x1, rn1 = _tc_post(xp, rn0, aggr_p[0], aggr_p[1], deg0, deg1,
                     Wp1_0[0:2], Wp1_0[2:], bp1_0r, Wp2_0.reshape(1, 16),
                     bp2_0r, Wn1, bn1r)

  (aggr_p1,) = _get_sc_kernel(False)(src_f, dst_f, et1, rn1, z32)
  x2, _ = _tc_post(x1, rn1, aggr_p1[0], aggr_p1[1], deg0, deg1,
                   Wp1_1[0:2], Wp1_1[2:], bp1_1r, Wp2_1.reshape(1, 16),
                   bp2_1r, Wn1, bn1r)
  return x2[:N]


# speedup vs baseline: 5.4262x; 1.4186x over previous
import functools

import jax
import jax.numpy as jnp
from jax import lax
from jax.experimental import pallas as pl
from jax.experimental.pallas import tpu as pltpu
from jax.experimental.pallas import tpu_sc as plsc

N = 50000
E = 1600000
HID = 32

NC = 2
NS = 16
NW = NC * NS

NP = 51200
NPT = NP // NS
B = 256
CHUNK = 128
NCH = B // CHUNK
EPW = 51200
EPAD = EPW * NW
NBLK = EPW // B


def _sc_layer_kernel(include_deg):
  mesh = plsc.VectorSubcoreMesh(core_axis_name="c", subcore_axis_name="s",
                                num_cores=NC, num_subcores=NS)

  out_type = [jax.ShapeDtypeStruct((NC, NP, HID), jnp.float32)]
  if include_deg:
    out_type.append(jax.ShapeDtypeStruct((NC, NP), jnp.float32))

  scratch = [
      pltpu.VMEM((NCH, CHUNK), jnp.int32),
      pltpu.VMEM((NCH, CHUNK), jnp.int32),
      pltpu.VMEM((B, HID), jnp.float32),
      pltpu.VMEM((B, HID), jnp.float32),
      pltpu.VMEM_SHARED((NP, HID), jnp.float32),
      pltpu.SemaphoreType.DMA,
      pltpu.SemaphoreType.DMA,
  ]
  if include_deg:
    scratch += [
        pltpu.VMEM((CHUNK,), jnp.float32),
        pltpu.VMEM_SHARED((NP,), jnp.float32),
        pltpu.SemaphoreType.DMA,
    ]

  def body(src2d, dst2d, eth, table, z32, *rest):
    if include_deg:
      (z1, aggr_out, deg_out, srcv, dstv, rows, erows,
       aggr_sh, gsem, ssem, onesv, deg_sh, dsem) = rest
    else:
      (aggr_out, srcv, dstv, rows, erows, aggr_sh, gsem, ssem) = rest

    c = lax.axis_index("c")
    s = lax.axis_index("s")
    wid = s * NC + c

    if include_deg:
      for i in range(CHUNK // 16):
        onesv[pl.ds(i * 16, 16)] = jnp.ones((16,), jnp.float32)

    pltpu.sync_copy(z32.at[pl.ds(s * NPT, NPT)],
                    aggr_sh.at[pl.ds(s * NPT, NPT)])
    if include_deg:
      pltpu.sync_copy(z1.at[pl.ds(s * NPT, NPT)],
                      deg_sh.at[pl.ds(s * NPT, NPT)])
    plsc.subcore_barrier()

    ebase = wid * EPW

    def block_body(blk, carry):
      base = pl.multiple_of(ebase + blk * B, B)
      row0 = pl.multiple_of(base // CHUNK, NCH)
      pltpu.sync_copy(src2d.at[pl.ds(row0, NCH)], srcv)
      pltpu.sync_copy(dst2d.at[pl.ds(row0, NCH)], dstv)

      ecp = pltpu.async_copy(eth.at[pl.ds(base, B)], erows, gsem)
      gcps = [
          pltpu.async_copy(table.at[srcv.at[j]],
                           rows.at[pl.ds(j * CHUNK, CHUNK)], gsem)
          for j in range(NCH)
      ]
      ecp.wait()
      for cp in gcps:
        cp.wait()

      scps = [
          pltpu.async_copy(rows.at[pl.ds(j * CHUNK, CHUNK)],
                           aggr_sh.at[dstv.at[j]], ssem, add=True)
          for j in range(NCH)
      ] + [
          pltpu.async_copy(erows.at[pl.ds(j * CHUNK, CHUNK)],
                           aggr_sh.at[dstv.at[j]], ssem, add=True)
          for j in range(NCH)
      ]
      for cp in scps:
        cp.wait()

      if include_deg:
        dcps = [
            pltpu.async_copy(onesv, deg_sh.at[dstv.at[j]], dsem, add=True)
            for j in range(NCH)
        ]
        for cp in dcps:
          cp.wait()
      return 0

    lax.fori_loop(0, NBLK, block_body, 0)

    plsc.subcore_barrier()

    pltpu.sync_copy(aggr_sh.at[pl.ds(s * NPT, NPT)],
                    aggr_out.at[c, pl.ds(s * NPT, NPT)])
    if include_deg:
      pltpu.sync_copy(deg_sh.at[pl.ds(s * NPT, NPT)],
                      deg_out.at[c, pl.ds(s * NPT, NPT)])

  return pl.kernel(body, out_type=out_type, mesh=mesh, scratch_types=scratch,
                   compiler_params=pltpu.CompilerParams(
                       needs_layout_passes=False,
                       use_tc_tiling_on_sc=False),
                   name="rgcn_sc_deg" if include_deg else "rgcn_sc")


@functools.lru_cache(maxsize=None)
def _get_sc_kernel(include_deg):
  return _sc_layer_kernel(include_deg)


RB = 1600
RBE = 6400


def _rb(minor, rb=RB):
  return pl.BlockSpec((rb, minor), lambda i: (i, 0))


def _full(shape):
  return pl.BlockSpec(shape, lambda i: tuple(0 for _ in shape))


def _tc_table_body(x_ref, wn_ref, bn_ref, out_ref):
  x = x_ref[...]
  wn = wn_ref[...]
  t = x[:, 0:1] * wn[0:1, :] + x[:, 1:2] * wn[1:2, :] + bn_ref[...]
  out_ref[...] = jnp.maximum(t, 0.0)


_tc_table = pl.pallas_call(
    _tc_table_body,
    grid=(NP // RB,),
    in_specs=[_rb(2), _full((2, HID)), _full((1, HID))],
    out_specs=_rb(HID),
    out_shape=jax.ShapeDtypeStruct((NP, HID), jnp.float32),
)


EROW = EPAD // 128
RBT = 64


def _tc_et_body(a0_ref, a1_ref, w0c0_ref, w0c1_ref, b0_ref,
                w1c0_ref, w1c1_ref, b1_ref, et0_ref, et1_ref):
  w0c0 = w0c0_ref[...]
  w0c1 = w0c1_ref[...]
  b0 = b0_ref[...]
  w1c0 = w1c0_ref[...]
  w1c1 = w1c1_ref[...]
  b1 = b1_ref[...]
  for r in range(RBT):
    a = a0_ref[r:r + 1, :]
    b = a1_ref[r:r + 1, :]
    t0 = jnp.maximum(a * w0c0 + b * w0c1 + b0, 0.0)
    et0_ref[r * 128:(r + 1) * 128, :] = jnp.swapaxes(t0, 0, 1)
    t1 = jnp.maximum(a * w1c0 + b * w1c1 + b1, 0.0)
    et1_ref[r * 128:(r + 1) * 128, :] = jnp.swapaxes(t1, 0, 1)


_tc_et = pl.pallas_call(
    _tc_et_body,
    grid=(EROW // RBT,),
    in_specs=[_rb(128, RBT), _rb(128, RBT),
              _full((HID, 1)), _full((HID, 1)), _full((HID, 1)),
              _full((HID, 1)), _full((HID, 1)), _full((HID, 1))],
    out_specs=[_rb(HID, RBT * 128), _rb(HID, RBT * 128)],
    out_shape=[
        jax.ShapeDtypeStruct((EPAD, HID), jnp.float32),
        jax.ShapeDtypeStruct((EPAD, HID), jnp.float32),
    ],
)


def _tc_post_body(x_ref, rn_ref, p0_ref, p1_ref, deg0_ref, deg1_ref,
                  wp1a_ref, wp1b_ref, bp1_ref, wp2_ref, bp2_ref,
                  wn_ref, bn_ref,
                  out_ref, rn_next_ref):
  x = x_ref[...]
  deg = deg0_ref[...] + deg1_ref[...]
  recip = 1.0 / jnp.maximum(deg, 1.0)
  aggr = (p0_ref[...] + p1_ref[...]) * recip
  t = aggr + rn_ref[...]
  wp1a = wp1a_ref[...]
  h = (x[:, 0:1] * wp1a[0:1, :] + x[:, 1:2] * wp1a[1:2, :]
       + jnp.dot(t, wp1b_ref[...], preferred_element_type=jnp.float32)
       + bp1_ref[...])
  h = jnp.maximum(h, 0.0)
  z = jnp.sum(h * wp2_ref[...], axis=1, keepdims=True)
  power = jax.nn.sigmoid(z + bp2_ref[...])
  out_ref[:, 0:1] = x[:, 0:1]
  out_ref[:, 1:2] = power
  wn = wn_ref[...]
  rn = x[:, 0:1] * wn[0:1, :] + power * wn[1:2, :] + bn_ref[...]
  rn_next_ref[...] = jnp.maximum(rn, 0.0)


_tc_post = pl.pallas_call(
    _tc_post_body,
    grid=(NP // RB,),
    in_specs=[_rb(2), _rb(HID), _rb(HID), _rb(HID), _rb(1), _rb(1),
              _full((2, 16)), _full((HID, 16)), _full((1, 16)),
              _full((1, 16)), _full((1, 1)),
              _full((2, HID)), _full((1, HID))],
    out_specs=[_rb(2), _rb(HID)],
    out_shape=[
        jax.ShapeDtypeStruct((NP, 2), jnp.float32),
        jax.ShapeDtypeStruct((NP, HID), jnp.float32),
    ],
)


def kernel(x, edge_index, edge_attr,
           Wn0, bn0, We0, be0, Wp1_0, bp1_0, Wp2_0, bp2_0,
           Wn1, bn1, We1, be1, Wp1_1, bp1_1, Wp2_1, bp2_1):
  src = edge_index[0]
  dst = edge_index[1]
  npad = EPAD - E
  pad_idx = (N + (jnp.arange(npad, dtype=jnp.int32) % (NP - N))).astype(jnp.int32)
  src_f = jnp.concatenate([src, pad_idx]).reshape(EPAD // CHUNK, CHUNK)
  dst_f = jnp.concatenate([dst, pad_idx]).reshape(EPAD // CHUNK, CHUNK)
  zpad = jnp.zeros((npad,), jnp.float32)
  a0p = jnp.concatenate([edge_attr[:, 0], zpad]).reshape(EROW, 128)
  a1p = jnp.concatenate([edge_attr[:, 1], zpad]).reshape(EROW, 128)
  xp = jnp.zeros((NP, 2), jnp.float32).at[:N].set(x)
  z32 = jnp.zeros((NP, HID), jnp.float32)
  z1 = jnp.zeros((NP,), jnp.float32)
  bn0r = bn0.reshape(1, HID)
  bn1r = bn1.reshape(1, HID)
  bp1_0r = bp1_0.reshape(1, 16)
  bp1_1r = bp1_1.reshape(1, 16)
  bp2_0r = bp2_0.reshape(1, 1)
  bp2_1r = bp2_1.reshape(1, 1)

  et0, et1 = _tc_et(a0p, a1p,
                    We0[0].reshape(HID, 1), We0[1].reshape(HID, 1),
                    be0.reshape(HID, 1),
                    We1[0].reshape(HID, 1), We1[1].reshape(HID, 1),
                    be1.reshape(HID, 1))
  rn0 = _tc_table(xp, Wn0, bn0r)

  aggr_p, deg_p = _get_sc_kernel(True)(src_f, dst_f, et0, rn0, z32, z1)
  deg0 = deg_p[0].reshape(NP, 1)
  deg1 = deg_p[1].reshape(NP, 1)
  x1, rn1 = _tc_post(xp, rn0, aggr_p[0], aggr_p[1], deg0, deg1,
                     Wp1_0[0:2], Wp1_0[2:], bp1_0r, Wp2_0.reshape(1, 16),
                     bp2_0r, Wn1, bn1r)

  (aggr_p1,) = _get_sc_kernel(False)(src_f, dst_f, et1, rn1, z32)
  x2, _ = _tc_post(x1, rn1, aggr_p1[0], aggr_p1[1], deg0, deg1,
                   Wp1_1[0:2], Wp1_1[2:], bp1_1r, Wp2_1.reshape(1, 16),
                   bp2_1r, Wn1, bn1r)
  return x2[:N]


# speedup vs baseline: 5.4987x; 1.0134x over previous
import functools

import jax
import jax.numpy as jnp
from jax import lax
from jax.experimental import pallas as pl
from jax.experimental.pallas import tpu as pltpu
from jax.experimental.pallas import tpu_sc as plsc

N = 50000
E = 1600000
HID = 32

NC = 2
NS = 16
NW = NC * NS

NP = 51200
NPT = NP // NS
B = 256
CHUNK = 128
NCH = B // CHUNK
EPW = 51200
EPAD = EPW * NW
NBLK = EPW // B


def _sc_layer_kernel(include_deg):
  mesh = plsc.VectorSubcoreMesh(core_axis_name="c", subcore_axis_name="s",
                                num_cores=NC, num_subcores=NS)

  out_type = [jax.ShapeDtypeStruct((NC, NP, HID), jnp.float32)]
  if include_deg:
    out_type.append(jax.ShapeDtypeStruct((NC, NP), jnp.float32))

  scratch = [
      pltpu.VMEM((NCH, CHUNK), jnp.int32),
      pltpu.VMEM((NCH, CHUNK), jnp.int32),
      pltpu.VMEM((B, HID), jnp.float32),
      pltpu.VMEM((B, HID), jnp.float32),
      pltpu.VMEM_SHARED((NP, HID), jnp.float32),
      pltpu.SemaphoreType.DMA,
      pltpu.SemaphoreType.DMA,
  ]
  if include_deg:
    scratch += [
        pltpu.VMEM((CHUNK,), jnp.float32),
        pltpu.VMEM_SHARED((NP,), jnp.float32),
        pltpu.SemaphoreType.DMA,
    ]

  def body(src2d, dst2d, eth, table, z32, *rest):
    if include_deg:
      (z1, aggr_out, deg_out, srcv, dstv, rows, erows,
       aggr_sh, gsem, ssem, onesv, deg_sh, dsem) = rest
    else:
      (aggr_out, srcv, dstv, rows, erows, aggr_sh, gsem, ssem) = rest

    c = lax.axis_index("c")
    s = lax.axis_index("s")
    wid = s * NC + c

    if include_deg:
      for i in range(CHUNK // 16):
        onesv[pl.ds(i * 16, 16)] = jnp.ones((16,), jnp.float32)

    pltpu.sync_copy(z32.at[pl.ds(s * NPT, NPT)],
                    aggr_sh.at[pl.ds(s * NPT, NPT)])
    if include_deg:
      pltpu.sync_copy(z1.at[pl.ds(s * NPT, NPT)],
                      deg_sh.at[pl.ds(s * NPT, NPT)])
    plsc.subcore_barrier()

    ebase = wid * EPW

    def block_body(blk, carry):
      base = pl.multiple_of(ebase + blk * B, B)
      row0 = pl.multiple_of(base // CHUNK, NCH)
      pltpu.sync_copy(src2d.at[pl.ds(row0, NCH)], srcv)
      pltpu.sync_copy(dst2d.at[pl.ds(row0, NCH)], dstv)

      ecp = pltpu.async_copy(eth.at[pl.ds(base, B)], erows, gsem)
      gcps = [
          pltpu.async_copy(table.at[srcv.at[j]],
                           rows.at[pl.ds(j * CHUNK, CHUNK)], gsem)
          for j in range(NCH)
      ]
      ecp.wait()
      for cp in gcps:
        cp.wait()

      scps = [
          pltpu.async_copy(rows.at[pl.ds(j * CHUNK, CHUNK)],
                           aggr_sh.at[dstv.at[j]], ssem, add=True)
          for j in range(NCH)
      ] + [
          pltpu.async_copy(erows.at[pl.ds(j * CHUNK, CHUNK)],
                           aggr_sh.at[dstv.at[j]], ssem, add=True)
          for j in range(NCH)
      ]
      for cp in scps:
        cp.wait()

      if include_deg:
        dcps = [
            pltpu.async_copy(onesv, deg_sh.at[dstv.at[j]], dsem, add=True)
            for j in range(NCH)
        ]
        for cp in dcps:
          cp.wait()
      return 0

    lax.fori_loop(0, NBLK, block_body, 0)

    plsc.subcore_barrier()

    pltpu.sync_copy(aggr_sh.at[pl.ds(s * NPT, NPT)],
                    aggr_out.at[c, pl.ds(s * NPT, NPT)])
    if include_deg:
      pltpu.sync_copy(deg_sh.at[pl.ds(s * NPT, NPT)],
                      deg_out.at[c, pl.ds(s * NPT, NPT)])

  return pl.kernel(body, out_type=out_type, mesh=mesh, scratch_types=scratch,
                   compiler_params=pltpu.CompilerParams(
                       needs_layout_passes=False,
                       use_tc_tiling_on_sc=False),
                   name="rgcn_sc_deg" if include_deg else "rgcn_sc")


@functools.lru_cache(maxsize=None)
def _get_sc_kernel(include_deg):
  return _sc_layer_kernel(include_deg)


RB = 1600
RBE = 6400


def _rb(minor, rb=RB):
  return pl.BlockSpec((rb, minor), lambda i: (i, 0))


def _full(shape):
  return pl.BlockSpec(shape, lambda i: tuple(0 for _ in shape))


def _tc_table_body(x_ref, wn_ref, bn_ref, out_ref):
  x = x_ref[...]
  wn = wn_ref[...]
  t = x[:, 0:1] * wn[0:1, :] + x[:, 1:2] * wn[1:2, :] + bn_ref[...]
  out_ref[...] = jnp.maximum(t, 0.0)


_tc_table = pl.pallas_call(
    _tc_table_body,
    grid=(NP // RB,),
    in_specs=[_rb(2), _full((2, HID)), _full((1, HID))],
    out_specs=_rb(HID),
    out_shape=jax.ShapeDtypeStruct((NP, HID), jnp.float32),
)


EROW = EPAD // 128
RBT = 64


def _tc_et_body(a0_ref, a1_ref, wc0_ref, wc1_ref, b_ref, et_ref):
  wc0 = wc0_ref[...]
  wc1 = wc1_ref[...]
  bb = b_ref[...]
  for r in range(RBT):
    a = a0_ref[r:r + 1, :]
    b = a1_ref[r:r + 1, :]
    t = jnp.maximum(a * wc0 + b * wc1 + bb, 0.0)
    et_ref[r * 128:(r + 1) * 128, :] = jnp.swapaxes(t, 0, 1)


_tc_et = pl.pallas_call(
    _tc_et_body,
    grid=(EROW // RBT,),
    in_specs=[_rb(128, RBT), _rb(128, RBT),
              _full((HID, 1)), _full((HID, 1)), _full((HID, 1))],
    out_specs=_rb(HID, RBT * 128),
    out_shape=jax.ShapeDtypeStruct((EPAD, HID), jnp.float32),
)


def _tc_post_body(x_ref, rn_ref, p0_ref, p1_ref, deg0_ref, deg1_ref,
                  wp1a_ref, wp1b_ref, bp1_ref, wp2_ref, bp2_ref,
                  wn_ref, bn_ref,
                  out_ref, rn_next_ref):
  x = x_ref[...]
  deg = deg0_ref[...] + deg1_ref[...]
  recip = 1.0 / jnp.maximum(deg, 1.0)
  aggr = (p0_ref[...] + p1_ref[...]) * recip
  t = aggr + rn_ref[...]
  wp1a = wp1a_ref[...]
  h = (x[:, 0:1] * wp1a[0:1, :] + x[:, 1:2] * wp1a[1:2, :]
       + jnp.dot(t, wp1b_ref[...], preferred_element_type=jnp.float32)
       + bp1_ref[...])
  h = jnp.maximum(h, 0.0)
  z = jnp.sum(h * wp2_ref[...], axis=1, keepdims=True)
  power = jax.nn.sigmoid(z + bp2_ref[...])
  out_ref[:, 0:1] = x[:, 0:1]
  out_ref[:, 1:2] = power
  wn = wn_ref[...]
  rn = x[:, 0:1] * wn[0:1, :] + power * wn[1:2, :] + bn_ref[...]
  rn_next_ref[...] = jnp.maximum(rn, 0.0)


_tc_post = pl.pallas_call(
    _tc_post_body,
    grid=(NP // RB,),
    in_specs=[_rb(2), _rb(HID), _rb(HID), _rb(HID), _rb(1), _rb(1),
              _full((2, 16)), _full((HID, 16)), _full((1, 16)),
              _full((1, 16)), _full((1, 1)),
              _full((2, HID)), _full((1, HID))],
    out_specs=[_rb(2), _rb(HID)],
    out_shape=[
        jax.ShapeDtypeStruct((NP, 2), jnp.float32),
        jax.ShapeDtypeStruct((NP, HID), jnp.float32),
    ],
)


def kernel(x, edge_index, edge_attr,
           Wn0, bn0, We0, be0, Wp1_0, bp1_0, Wp2_0, bp2_0,
           Wn1, bn1, We1, be1, Wp1_1, bp1_1, Wp2_1, bp2_1):
  src = edge_index[0]
  dst = edge_index[1]
  npad = EPAD - E
  pad_idx = (N + (jnp.arange(npad, dtype=jnp.int32) % (NP - N))).astype(jnp.int32)
  src_f = jnp.concatenate([src, pad_idx]).reshape(EPAD // CHUNK, CHUNK)
  dst_f = jnp.concatenate([dst, pad_idx]).reshape(EPAD // CHUNK, CHUNK)
  zpad = jnp.zeros((npad,), jnp.float32)
  a0p = jnp.concatenate([edge_attr[:, 0], zpad]).reshape(EROW, 128)
  a1p = jnp.concatenate([edge_attr[:, 1], zpad]).reshape(EROW, 128)
  xp = jnp.zeros((NP, 2), jnp.float32).at[:N].set(x)
  z32 = jnp.zeros((NP, HID), jnp.float32)
  z1 = jnp.zeros((NP,), jnp.float32)
  bn0r = bn0.reshape(1, HID)
  bn1r = bn1.reshape(1, HID)
  bp1_0r = bp1_0.reshape(1, 16)
  bp1_1r = bp1_1.reshape(1, 16)
  bp2_0r = bp2_0.reshape(1, 1)
  bp2_1r = bp2_1.reshape(1, 1)

  et0 = _tc_et(a0p, a1p, We0[0].reshape(HID, 1), We0[1].reshape(HID, 1),
               be0.reshape(HID, 1))
  rn0 = _tc_table(xp, Wn0, bn0r)

  aggr_p, deg_p = _get_sc_kernel(True)(src_f, dst_f, et0, rn0, z32, z1)
  et1 = _tc_et(a0p, a1p, We1[0].reshape(HID, 1), We1[1].reshape(HID, 1),
               be1.reshape(HID, 1))
  deg0 = deg_p[0].reshape(NP, 1)
  deg1 = deg_p[1].reshape(NP, 1)
  x1, rn1 = _tc_post(xp, rn0, aggr_p[0], aggr_p[1], deg0, deg1,
                     Wp1_0[0:2], Wp1_0[2:], bp1_0r, Wp2_0.reshape(1, 16),
                     bp2_0r, Wn1, bn1r)

  (aggr_p1,) = _get_sc_kernel(False)(src_f, dst_f, et1, rn1, z32)
  x2, _ = _tc_post(x1, rn1, aggr_p1[0], aggr_p1[1], deg0, deg1,
                   Wp1_1[0:2], Wp1_1[2:], bp1_1r, Wp2_1.reshape(1, 16),
                   bp2_1r, Wn1, bn1r)
  return x2[:N]
